# baseline (speedup 1.0000x reference)
import functools

import jax
import jax.numpy as jnp
from jax import lax
from jax.experimental import pallas as pl
from jax.experimental.pallas import tpu as pltpu
from jax.experimental.pallas import tpu_sc as plsc

_NUM_CHANNELS = 384
_NUM_BATCH = 16
_ROW = 56 * 56
_NROWS = _NUM_BATCH * _NUM_CHANNELS
_NC = 2
_NS = 16
_NW = _NC * _NS
_ROWS_PER_W = _NROWS // _NW
_CHUNK = 8
_NCHUNK = _ROWS_PER_W // _CHUNK
_NBUF = 4
_LOOK = 2
_LANES = 16


def _build_shuffle():
    mesh = plsc.VectorSubcoreMesh(core_axis_name="c", subcore_axis_name="s")

    @functools.partial(
        pl.kernel,
        mesh=mesh,
        out_type=jax.ShapeDtypeStruct((_NROWS, _ROW), jnp.float32),
        compiler_params=pltpu.CompilerParams(use_tc_tiling_on_sc=False),
        scratch_types=[
            pltpu.VMEM((_ROWS_PER_W,), jnp.int32),
            [pltpu.VMEM((_CHUNK, _ROW), jnp.float32) for _ in range(_NBUF)],
            [pltpu.SemaphoreType.DMA for _ in range(_NBUF)],
            [pltpu.SemaphoreType.DMA for _ in range(_NBUF)],
        ],
    )
    def shuffle(x_hbm, idx_hbm, out_hbm, idx_v, bufs, gsems, ssems):
        wid = lax.axis_index("s") * _NC + lax.axis_index("c")
        batch = wid // 2
        c0 = (wid % 2) * _ROWS_PER_W
        row_base = wid * _ROWS_PER_W

        pltpu.sync_copy(idx_hbm.at[pl.ds(c0, _ROWS_PER_W)], idx_v)
        off = batch * _NUM_CHANNELS
        for j in range(_ROWS_PER_W // _LANES):
            sl = pl.ds(j * _LANES, _LANES)
            idx_v[sl] = idx_v[sl] + off

        def gather(k):
            return pltpu.async_copy(
                x_hbm.at[idx_v.at[pl.ds(k * _CHUNK, _CHUNK)]],
                bufs[k % _NBUF], gsems[k % _NBUF])

        gh = [None] * _NCHUNK
        sh = [None] * _NCHUNK
        for k in range(_LOOK):
            gh[k] = gather(k)
        for k in range(_NCHUNK):
            p = k % _NBUF
            gh[k].wait()
            sh[k] = pltpu.async_copy(
                bufs[p],
                out_hbm.at[pl.ds(row_base + k * _CHUNK, _CHUNK)],
                ssems[p])
            nxt = k + _LOOK
            if nxt < _NCHUNK:
                drain = nxt - _NBUF
                if drain >= 0:
                    sh[drain].wait()
                gh[nxt] = gather(nxt)
        for k in range(max(0, _NCHUNK - _NBUF), _NCHUNK):
            sh[k].wait()

    return shuffle


_shuffle = _build_shuffle()


def kernel(x, objective, indices, rev_indices):
    table = x.reshape(_NROWS, _ROW)
    out = _shuffle(table, indices)
    return (out.reshape(x.shape), objective)


# speedup vs baseline: 10.2940x; 10.2940x over previous
import jax
import jax.numpy as jnp
from jax import lax
from jax.experimental import pallas as pl
from jax.experimental.pallas import tpu as pltpu

_NUM_CHANNELS = 384
_NUM_BATCH = 16
_IMG = 56
_NPIX = _NUM_BATCH * _IMG * _IMG
_BLK = 3136
_GRID = _NPIX // _BLK


def _mm_body(idx_ref, a_ref, out_ref):
    iota = lax.broadcasted_iota(jnp.int32, (_NUM_CHANNELS, _NUM_CHANNELS), 0)
    onehot = (iota == jnp.broadcast_to(
        idx_ref[...], (_NUM_CHANNELS, _NUM_CHANNELS))).astype(jnp.float32)
    out_ref[...] = jnp.dot(a_ref[...], onehot,
                           preferred_element_type=jnp.float32)


def _permute_mm(a, indices):
    return pl.pallas_call(
        _mm_body,
        grid=(_GRID,),
        in_specs=[
            pl.BlockSpec((1, _NUM_CHANNELS), lambda i: (0, 0)),
            pl.BlockSpec((_BLK, _NUM_CHANNELS), lambda i: (i, 0)),
        ],
        out_specs=pl.BlockSpec((_BLK, _NUM_CHANNELS), lambda i: (i, 0)),
        out_shape=jax.ShapeDtypeStruct((_NPIX, _NUM_CHANNELS), jnp.float32),
        compiler_params=pltpu.CompilerParams(
            dimension_semantics=("arbitrary",)),
    )(indices.reshape(1, _NUM_CHANNELS), a)


def kernel(x, objective, indices, rev_indices):
    a = x.transpose(0, 2, 3, 1).reshape(_NPIX, _NUM_CHANNELS)
    out = _permute_mm(a, indices)
    out = out.reshape(_NUM_BATCH, _IMG, _IMG, _NUM_CHANNELS)
    return (out.transpose(0, 3, 1, 2), objective)
